# baseline (speedup 1.0000x reference)
import functools

import numpy as np
import jax
import jax.numpy as jnp
from jax import lax
from jax.experimental import pallas as pl
from jax.experimental.pallas import tpu as pltpu
from jax.experimental.pallas import tpu_sc as plsc

_MAX_LEN = 5000
_NC = 2
_NS = 16
_NW = _NC * _NS
_L = 16


def _positional_encoding(max_len, d_model):
    pos = np.arange(max_len)[:, None].astype(np.float32)
    i = np.arange(d_model)[None, :].astype(np.float32)
    angle_rates = 1.0 / np.power(10000.0, (2.0 * np.floor(i / 2.0)) / np.float32(d_model))
    angle_rads = pos * angle_rates
    angle_rads[:, 0::2] = np.sin(angle_rads[:, 0::2])
    angle_rads[:, 1::2] = np.cos(angle_rads[:, 1::2])
    return angle_rads


def _splat(v, dtype=jnp.int32):
    return jnp.full((_L,), v, dtype=dtype)


_TBLK = 512


def _make_table(Et, V, D, scale):
    def body(x_ref, o_ref):
        xt = x_ref[...].T * scale
        o_ref[...] = jnp.concatenate(
            [xt, jnp.zeros((_TBLK, 128 - D), jnp.float32)], axis=1)

    return pl.pallas_call(
        body,
        grid=(pl.cdiv(V, _TBLK),),
        in_specs=[pl.BlockSpec((D, _TBLK), lambda i: (0, i))],
        out_specs=pl.BlockSpec((_TBLK, 128), lambda i: (i, 0)),
        out_shape=jax.ShapeDtypeStruct((V, 128), jnp.float32),
    )(Et)


@functools.partial(jax.jit, static_argnums=(3, 4, 5))
def _launch(inp, E, pos, B, S, D):
    BW = B // _NW
    V = E.shape[0]
    scale = float(np.float32(np.sqrt(np.float32(D))))
    R8 = D // 8
    CB = B // 128

    table = _make_table(E.T, V, D, scale)

    mesh = plsc.VectorSubcoreMesh(
        core_axis_name="c", subcore_axis_name="s",
        num_cores=_NC, num_subcores=_NS)

    @functools.partial(
        pl.kernel,
        out_type=jax.ShapeDtypeStruct((S, R8, CB, 8, 128), jnp.float32),
        mesh=mesh,
        scratch_types=[
            pltpu.VMEM((BW, S), jnp.int32),
            pltpu.VMEM((S, D), jnp.float32),
            pltpu.VMEM((128,), jnp.int32),
            pltpu.VMEM((128,), jnp.int32),
            pltpu.VMEM((128, 128), jnp.float32),
            pltpu.VMEM((128, 128), jnp.float32),
            pltpu.VMEM((R8, 8, 129), jnp.float32),
            pltpu.VMEM((R8, 8, 129), jnp.float32),
            pltpu.SemaphoreType.DMA,
            pltpu.SemaphoreType.DMA,
            pltpu.SemaphoreType.DMA,
            pltpu.SemaphoreType.DMA,
        ],
        compiler_params=pltpu.CompilerParams(
            use_tc_tiling_on_sc=False, needs_layout_passes=False),
    )
    def run(inp_hbm, table_hbm, pos_hbm, out_hbm,
            idxblk, pos_v, idxA, idxB, rowsA, rowsB, outA, outB,
            gsemA, gsemB, wsemA, wsemB):
        wid = lax.axis_index("s") * _NC + lax.axis_index("c")
        pltpu.sync_copy(inp_hbm.at[pl.ds(wid * BW, BW), :], idxblk)
        pltpu.sync_copy(pos_hbm, pos_v)
        iota = lax.iota(jnp.int32, _L)

        def build_idx(s, idx_ref):
            for k in range(BW // _L):
                v = plsc.load_gather(idxblk, [iota + (k * _L), _splat(s)])
                idx_ref[pl.ds(k * _L, _L)] = v

        dch = D // _L
        r_tile = [(iota + c * _L) // 8 for c in range(dch)]
        r_sub = [(iota + c * _L) % 8 for c in range(dch)]

        def compute(s, rows_ref, out_ref):
            posv = [pos_v[s, pl.ds(c * _L, _L)] for c in range(dch)]

            @plsc.parallel_loop(0, BW, 1, unroll=8)
            def bbody(b):
                bs = _splat(b)
                for c in range(dch):
                    g = rows_ref[b, pl.ds(c * _L, _L)]
                    v = g + posv[c]
                    plsc.store_scatter(out_ref, [r_tile[c], r_sub[c], bs], v)

        def fire_gather(idx_ref, rows_ref, sem):
            return pltpu.async_copy(table_hbm.at[idx_ref], rows_ref, sem)

        def fire_wb(s, out_ref, sem):
            return pltpu.async_copy(
                out_ref.at[:, :, pl.ds(0, 128)], out_hbm.at[s, :, wid], sem)

        def drain_gather(idx_ref, rows_ref, sem):
            pltpu.make_async_copy(table_hbm.at[idx_ref], rows_ref, sem).wait()

        def drain_wb(out_ref, sem):
            pltpu.make_async_copy(
                out_ref.at[:, :, pl.ds(0, 128)], out_hbm.at[0, :, wid], sem).wait()

        build_idx(0, idxA)
        fire_gather(idxA, rowsA, gsemA)

        def sbody(i, carry):
            s0 = 2 * i
            s1 = s0 + 1
            build_idx(s1, idxB)
            fire_gather(idxB, rowsB, gsemB)
            drain_gather(idxA, rowsA, gsemA)
            @pl.when(i > 0)
            def _():
                drain_wb(outA, wsemA)
            compute(s0, rowsA, outA)
            fire_wb(s0, outA, wsemA)
            @pl.when(i < (S // 2 - 1))
            def _():
                build_idx(s0 + 2, idxA)
                fire_gather(idxA, rowsA, gsemA)
            drain_gather(idxB, rowsB, gsemB)
            @pl.when(i > 0)
            def _():
                drain_wb(outB, wsemB)
            compute(s1, rowsB, outB)
            fire_wb(s1, outB, wsemB)
            return carry
        lax.fori_loop(0, S // 2, sbody, 0)

        drain_wb(outA, wsemA)
        drain_wb(outB, wsemB)

    return run(inp, table, pos)


def kernel(input, E):
    B, S = input.shape
    V, D = E.shape
    pos = jnp.asarray(_positional_encoding(_MAX_LEN, D)[:S], dtype=jnp.float32)
    out5 = _launch(input, E, pos, B, S, D)
    return out5.transpose(2, 4, 0, 1, 3).reshape(B, S, D)


# speedup vs baseline: 1.8258x; 1.8258x over previous
import functools

import numpy as np
import jax
import jax.numpy as jnp
from jax import lax
from jax.experimental import pallas as pl
from jax.experimental.pallas import tpu as pltpu
from jax.experimental.pallas import tpu_sc as plsc

_MAX_LEN = 5000
_NC = 2
_NS = 16
_NW = _NC * _NS
_L = 16


def _positional_encoding(max_len, d_model):
    pos = np.arange(max_len)[:, None].astype(np.float32)
    i = np.arange(d_model)[None, :].astype(np.float32)
    angle_rates = 1.0 / np.power(10000.0, (2.0 * np.floor(i / 2.0)) / np.float32(d_model))
    angle_rads = pos * angle_rates
    angle_rads[:, 0::2] = np.sin(angle_rads[:, 0::2])
    angle_rads[:, 1::2] = np.cos(angle_rads[:, 1::2])
    return angle_rads


def _splat(v, dtype=jnp.int32):
    return jnp.full((_L,), v, dtype=dtype)


_TBLK = 512


def _make_table(Et, V, D, scale):
    def body(x_ref, o_ref):
        xt = x_ref[...].T * scale
        o_ref[...] = jnp.concatenate(
            [xt, jnp.zeros((_TBLK, 128 - D), jnp.float32)], axis=1)

    return pl.pallas_call(
        body,
        grid=(pl.cdiv(V, _TBLK),),
        in_specs=[pl.BlockSpec((D, _TBLK), lambda i: (0, i))],
        out_specs=pl.BlockSpec((_TBLK, 128), lambda i: (i, 0)),
        out_shape=jax.ShapeDtypeStruct((V, 128), jnp.float32),
    )(Et)


@functools.partial(jax.jit, static_argnums=(3, 4, 5))
def _launch(inp, E, pos, B, S, D):
    BW = B // _NW
    V = E.shape[0]
    scale = float(np.float32(np.sqrt(np.float32(D))))
    R8 = D // 8
    CB = B // 128

    mesh = plsc.VectorSubcoreMesh(
        core_axis_name="c", subcore_axis_name="s",
        num_cores=_NC, num_subcores=_NS)

    @functools.partial(
        pl.kernel,
        out_type=jax.ShapeDtypeStruct((S, R8, CB, 8, 128), jnp.float32),
        mesh=mesh,
        scratch_types=[
            pltpu.VMEM((BW, S), jnp.int32),
            pltpu.VMEM((S, D), jnp.float32),
            pltpu.VMEM((128,), jnp.int32),
            pltpu.VMEM((128,), jnp.int32),
            pltpu.VMEM((128, D), jnp.float32),
            pltpu.VMEM((128, D), jnp.float32),
            pltpu.VMEM((R8, 8, 129), jnp.float32),
            pltpu.VMEM((R8, 8, 129), jnp.float32),
            pltpu.SemaphoreType.DMA,
            pltpu.SemaphoreType.DMA,
            pltpu.SemaphoreType.DMA,
            pltpu.SemaphoreType.DMA,
        ],
        compiler_params=pltpu.CompilerParams(
            use_tc_tiling_on_sc=False, needs_layout_passes=False),
    )
    def run(inp_hbm, table_hbm, pos_hbm, out_hbm,
            idxblk, pos_v, idxA, idxB, rowsA, rowsB, outA, outB,
            gsemA, gsemB, wsemA, wsemB):
        wid = lax.axis_index("s") * _NC + lax.axis_index("c")
        pltpu.sync_copy(inp_hbm.at[pl.ds(wid * BW, BW), :], idxblk)
        pltpu.sync_copy(pos_hbm, pos_v)
        iota = lax.iota(jnp.int32, _L)

        def build_idx(s, idx_ref):
            for k in range(BW // _L):
                v = plsc.load_gather(idxblk, [iota + (k * _L), _splat(s)])
                idx_ref[pl.ds(k * _L, _L)] = v

        dch = D // _L
        r_tile = [(iota + c * _L) // 8 for c in range(dch)]
        r_sub = [(iota + c * _L) % 8 for c in range(dch)]

        def compute(s, rows_ref, out_ref):
            posv = [pos_v[s, pl.ds(c * _L, _L)] for c in range(dch)]

            @plsc.parallel_loop(0, BW, 1, unroll=8)
            def bbody(b):
                bs = _splat(b)
                for c in range(dch):
                    g = rows_ref[b, pl.ds(c * _L, _L)]
                    v = g * scale + posv[c]
                    plsc.store_scatter(out_ref, [r_tile[c], r_sub[c], bs], v)

        def fire_gather(idx_ref, rows_ref, sem):
            return pltpu.async_copy(table_hbm.at[idx_ref], rows_ref, sem)

        def fire_wb(s, out_ref, sem):
            return pltpu.async_copy(
                out_ref.at[:, :, pl.ds(0, 128)], out_hbm.at[s, :, wid], sem)

        def drain_gather(idx_ref, rows_ref, sem):
            pltpu.make_async_copy(table_hbm.at[idx_ref], rows_ref, sem).wait()

        def drain_wb(out_ref, sem):
            pltpu.make_async_copy(
                out_ref.at[:, :, pl.ds(0, 128)], out_hbm.at[0, :, wid], sem).wait()

        build_idx(0, idxA)
        fire_gather(idxA, rowsA, gsemA)

        def sbody(i, carry):
            s0 = 2 * i
            s1 = s0 + 1
            build_idx(s1, idxB)
            fire_gather(idxB, rowsB, gsemB)
            drain_gather(idxA, rowsA, gsemA)
            @pl.when(i > 0)
            def _():
                drain_wb(outA, wsemA)
            compute(s0, rowsA, outA)
            fire_wb(s0, outA, wsemA)
            @pl.when(i < (S // 2 - 1))
            def _():
                build_idx(s0 + 2, idxA)
                fire_gather(idxA, rowsA, gsemA)
            drain_gather(idxB, rowsB, gsemB)
            @pl.when(i > 0)
            def _():
                drain_wb(outB, wsemB)
            compute(s1, rowsB, outB)
            fire_wb(s1, outB, wsemB)
            return carry
        lax.fori_loop(0, S // 2, sbody, 0)

        drain_wb(outA, wsemA)
        drain_wb(outB, wsemB)

    return run(inp, E, pos)


def kernel(input, E):
    B, S = input.shape
    V, D = E.shape
    pos = jnp.asarray(_positional_encoding(_MAX_LEN, D)[:S], dtype=jnp.float32)
    out5 = _launch(input, E, pos, B, S, D)
    return out5.transpose(2, 4, 0, 1, 3).reshape(B, S, D)


# speedup vs baseline: 1.9118x; 1.0471x over previous
import functools

import numpy as np
import jax
import jax.numpy as jnp
from jax import lax
from jax.experimental import pallas as pl
from jax.experimental.pallas import tpu as pltpu
from jax.experimental.pallas import tpu_sc as plsc

_MAX_LEN = 5000
_NC = 2
_NS = 16
_NW = _NC * _NS
_L = 16


def _positional_encoding(max_len, d_model):
    pos = np.arange(max_len)[:, None].astype(np.float32)
    i = np.arange(d_model)[None, :].astype(np.float32)
    angle_rates = 1.0 / np.power(10000.0, (2.0 * np.floor(i / 2.0)) / np.float32(d_model))
    angle_rads = pos * angle_rates
    angle_rads[:, 0::2] = np.sin(angle_rads[:, 0::2])
    angle_rads[:, 1::2] = np.cos(angle_rads[:, 1::2])
    return angle_rads


def _splat(v, dtype=jnp.int32):
    return jnp.full((_L,), v, dtype=dtype)


@functools.partial(jax.jit, static_argnums=(3, 4, 5))
def _launch(inp, E, pos, B, S, D):
    BW = B // _NW
    V = E.shape[0]
    scale = float(np.float32(np.sqrt(np.float32(D))))
    R8 = D // 8
    CB = B // 128

    mesh = plsc.VectorSubcoreMesh(
        core_axis_name="c", subcore_axis_name="s",
        num_cores=_NC, num_subcores=_NS)

    @functools.partial(
        pl.kernel,
        out_type=jax.ShapeDtypeStruct((S, R8, CB, 8, 128), jnp.float32),
        mesh=mesh,
        scratch_types=[
            pltpu.VMEM((BW, S), jnp.int32),
            pltpu.VMEM((S, D), jnp.float32),
            pltpu.VMEM((256,), jnp.int32),
            pltpu.VMEM((256,), jnp.int32),
            pltpu.VMEM((256, D), jnp.float32),
            pltpu.VMEM((256, D), jnp.float32),
            pltpu.VMEM((2, R8, 8, 129), jnp.float32),
            pltpu.VMEM((2, R8, 8, 129), jnp.float32),
            pltpu.SemaphoreType.DMA,
            pltpu.SemaphoreType.DMA,
            pltpu.SemaphoreType.DMA,
            pltpu.SemaphoreType.DMA,
        ],
        compiler_params=pltpu.CompilerParams(
            use_tc_tiling_on_sc=False, needs_layout_passes=False),
    )
    def run(inp_hbm, table_hbm, pos_hbm, out_hbm,
            idxblk, pos_v, idxA, idxB, rowsA, rowsB, outA, outB,
            gsemA, gsemB, wsemA, wsemB):
        wid = lax.axis_index("s") * _NC + lax.axis_index("c")
        pltpu.sync_copy(inp_hbm.at[pl.ds(wid * BW, BW), :], idxblk)
        pltpu.sync_copy(pos_hbm, pos_v)
        iota = lax.iota(jnp.int32, _L)

        def build_idx(g, idx_ref):
            sg = 2 * g
            for j in range(2):
                for k in range(BW // _L):
                    v = plsc.load_gather(idxblk, [iota + (k * _L), _splat(sg + j)])
                    idx_ref[pl.ds(j * BW + k * _L, _L)] = v

        dch = D // _L
        r_tile = [(iota + c * _L) // 8 for c in range(dch)]
        r_sub = [(iota + c * _L) % 8 for c in range(dch)]

        def compute(s, rows_ref, out_ref):
            posv = [pos_v[s, pl.ds(c * _L, _L)] for c in range(dch)]

            @plsc.parallel_loop(0, BW, 1, unroll=8)
            def bbody(b):
                bs = _splat(b)
                for c in range(dch):
                    g = rows_ref[b, pl.ds(c * _L, _L)]
                    v = g * scale + posv[c]
                    plsc.store_scatter(out_ref, [r_tile[c], r_sub[c], bs], v)

        def fire_gather(idx_ref, rows_ref, sem):
            return pltpu.async_copy(table_hbm.at[idx_ref], rows_ref, sem)

        def fire_wb(s, out_ref, sem):
            return pltpu.async_copy(
                out_ref.at[:, :, :, pl.ds(0, 128)],
                out_hbm.at[pl.ds(s, 2), :, wid], sem)

        def drain_gather(idx_ref, rows_ref, sem):
            pltpu.make_async_copy(table_hbm.at[idx_ref], rows_ref, sem).wait()

        def drain_wb(out_ref, sem):
            pltpu.make_async_copy(
                out_ref.at[:, :, :, pl.ds(0, 128)],
                out_hbm.at[pl.ds(0, 2), :, wid], sem).wait()

        build_idx(0, idxA)
        fire_gather(idxA, rowsA, gsemA)
        NG = S // 2

        def sbody(i, carry):
            g0 = 2 * i
            g1 = g0 + 1
            s0 = 2 * g0
            build_idx(g1, idxB)
            fire_gather(idxB, rowsB, gsemB)
            drain_gather(idxA, rowsA, gsemA)
            @pl.when(i > 0)
            def _():
                drain_wb(outA, wsemA)
            compute(s0, rowsA.at[pl.ds(0, BW), :], outA.at[0])
            compute(s0 + 1, rowsA.at[pl.ds(BW, BW), :], outA.at[1])
            fire_wb(s0, outA, wsemA)
            @pl.when(i < (NG // 2 - 1))
            def _():
                build_idx(g0 + 2, idxA)
                fire_gather(idxA, rowsA, gsemA)
            drain_gather(idxB, rowsB, gsemB)
            @pl.when(i > 0)
            def _():
                drain_wb(outB, wsemB)
            compute(s0 + 2, rowsB.at[pl.ds(0, BW), :], outB.at[0])
            compute(s0 + 3, rowsB.at[pl.ds(BW, BW), :], outB.at[1])
            fire_wb(s0 + 2, outB, wsemB)
            return carry
        lax.fori_loop(0, NG // 2, sbody, 0)

        drain_wb(outA, wsemA)
        drain_wb(outB, wsemB)

    return run(inp, E, pos)


def kernel(input, E):
    B, S = input.shape
    V, D = E.shape
    pos = jnp.asarray(_positional_encoding(_MAX_LEN, D)[:S], dtype=jnp.float32)
    out5 = _launch(input, E, pos, B, S, D)
    return out5.transpose(2, 4, 0, 1, 3).reshape(B, S, D)
